# baseline (speedup 1.0000x reference)
import functools

import jax
import jax.numpy as jnp
from jax import lax
from jax.experimental import pallas as pl
from jax.experimental.pallas import tpu as pltpu
from jax.experimental.pallas import tpu_sc as plsc

NCLASS = 19
NPIX = 4 * 512 * 512
BINS = 256
NW = 32
SHARD = NPIX // NW
RW = 16
SROW = 2048 // NW
NSUB = 4
KEYS2 = 2 * BINS
LHIST = 16 * KEYS2


def _keys_body(x_ref, t_ref, out_ref):
    x = x_ref[0]
    m = jnp.max(x, axis=0, keepdims=True)
    e = jnp.exp(x - m)
    p = e / jnp.sum(e, axis=0, keepdims=True)
    t = t_ref[...]
    cls = lax.broadcasted_iota(jnp.int32, (NCLASS, 1, 1), 0)
    gt = t == cls
    a = jnp.where(gt, 1.0 - p, p)
    b = jnp.minimum((a * BINS).astype(jnp.int32), BINS - 1)
    out_ref[...] = jnp.where(gt, b + BINS, b).astype(jnp.int16)


def _keys(x, t):
    return pl.pallas_call(
        _keys_body,
        grid=(4, 512 // RW),
        in_specs=[
            pl.BlockSpec((1, NCLASS, RW, 512), lambda b, j: (b, 0, j, 0)),
            pl.BlockSpec((1, RW, 512), lambda b, j: (b, j, 0)),
        ],
        out_specs=pl.BlockSpec(
            (NCLASS, RW, 512), lambda b, j: (0, b * (512 // RW) + j, 0)),
        out_shape=jax.ShapeDtypeStruct((NCLASS, 2048, 512), jnp.int16),
        compiler_params=pltpu.CompilerParams(
            dimension_semantics=("parallel", "parallel")),
    )(x, t)


def _sc_hist_body(keys_hbm, hist_hbm, kb0, kb1, hist16, red, sem0, sem1):
    wid = lax.axis_index("s") * 2 + lax.axis_index("c")
    base = wid * SROW
    lane_off = lax.broadcasted_iota(jnp.int32, (16,), 0) * KEYS2
    ones = jnp.ones((16,), jnp.int32)
    zeros = jnp.zeros((16,), jnp.int32)

    def start(c, kb, sem):
        pltpu.async_copy(keys_hbm.at[c, pl.ds(base, SROW)], kb, sem)

    def wait(c, kb, sem):
        pltpu.make_async_copy(keys_hbm.at[c, pl.ds(base, SROW)], kb, sem).wait()

    def process(c, kb):
        def zero_step(i, _):
            for u in range(16):
                hist16[pl.ds((i * 16 + u) * 16, 16)] = zeros
            return ()
        lax.fori_loop(0, NSUB * LHIST // (16 * 16), zero_step, ())

        def scat_row(r, _):
            def scat_step(i, _):
                pairs = [plsc.bitcast(kb[r, pl.ds((i * 4 + u) * 32, 32)],
                                      jnp.int32) for u in range(4)]
                ks = []
                for pv in pairs:
                    ks.append(pv & 0xFFFF)
                    ks.append(lax.shift_right_logical(pv, 16))
                idxs = [ks[u] + lane_off + (u % NSUB) * LHIST
                        for u in range(8)]
                for u in range(8):
                    plsc.addupdate_scatter(hist16, [idxs[u]], ones)
                return ()
            lax.fori_loop(0, 512 // (16 * 8), scat_step, ())
            return ()
        lax.fori_loop(0, SROW, scat_row, ())

        def red_step(j, _):
            vals = [hist16[pl.ds(t * KEYS2 + j * 16, 16)]
                    for t in range(NSUB * 16)]
            while len(vals) > 1:
                vals = [vals[t] + vals[t + 1] for t in range(0, len(vals), 2)]
            red[pl.ds(j * 16, 16)] = vals[0]
            return ()
        lax.fori_loop(0, KEYS2 // 16, red_step, ())

        pltpu.sync_copy(red, hist_hbm.at[c, wid])

    start(0, kb0, sem0)

    def per_class(c, _):
        @pl.when(c % 2 == 0)
        def _():
            wait(c, kb0, sem0)
            @pl.when(c + 1 < NCLASS)
            def _():
                start(c + 1, kb1, sem1)
            process(c, kb0)

        @pl.when(c % 2 == 1)
        def _():
            wait(c, kb1, sem1)
            @pl.when(c + 1 < NCLASS)
            def _():
                start(c + 1, kb0, sem0)
            process(c, kb1)

        return ()

    lax.fori_loop(0, NCLASS, per_class, ())


@functools.lru_cache(maxsize=1)
def _sc_hist_kernel():
    return pl.kernel(
        _sc_hist_body,
        mesh=plsc.VectorSubcoreMesh(core_axis_name="c", subcore_axis_name="s"),
        out_type=jax.ShapeDtypeStruct((NCLASS, NW, KEYS2), jnp.int32),
        scratch_types=[
            pltpu.VMEM((SROW, 512), jnp.int16),
            pltpu.VMEM((SROW, 512), jnp.int16),
            pltpu.VMEM((NSUB * LHIST,), jnp.int32),
            pltpu.VMEM((KEYS2,), jnp.int32),
            pltpu.SemaphoreType.DMA,
            pltpu.SemaphoreType.DMA,
        ],
        compiler_params=pltpu.CompilerParams(needs_layout_passes=False),
    )


def _sc_hist(keys):
    return _sc_hist_kernel()(keys)


def _strict_tri(n, lower=False):
    r = lax.broadcasted_iota(jnp.int32, (n, n), 0)
    col = lax.broadcasted_iota(jnp.int32, (n, n), 1)
    return ((r > col) if lower else (r < col)).astype(jnp.float32)


def _finish_body(hist_ref, out_ref):
    h = hist_ref[...].astype(jnp.float32)
    n2 = jnp.sum(h, axis=1)
    nn = n2[:, :BINS] + n2[:, BINS:]
    pp = n2[:, BINS:]
    tri = _strict_tri(BINS)
    aex = jnp.dot(nn, tri, preferred_element_type=jnp.float32)
    pex = jnp.dot(pp, tri, preferred_element_type=jnp.float32)
    nc = jnp.sum(nn, axis=1, keepdims=True)
    g = jnp.sum(pp, axis=1, keepdims=True)
    k = nc - aex
    s = g - pex
    u = g + k - s
    j = jnp.where(k > 0.5, 1.0 - (g - s) / jnp.maximum(u, 1.0), 0.0)
    val = (jnp.sum(j) - 0.5 * NCLASS) / (BINS * NCLASS)
    out_ref[...] = val.reshape(1, 1)


def _finish(hist):
    return pl.pallas_call(
        _finish_body,
        out_shape=jax.ShapeDtypeStruct((1, 1), jnp.float32),
    )(hist)


def kernel(input, target):
    keys = _keys(input, target)
    hist = _sc_hist(keys)
    return _finish(hist)[0, 0]


# speedup vs baseline: 1.0437x; 1.0437x over previous
import functools

import jax
import jax.numpy as jnp
from jax import lax
from jax.experimental import pallas as pl
from jax.experimental.pallas import tpu as pltpu
from jax.experimental.pallas import tpu_sc as plsc

NCLASS = 19
NPIX = 4 * 512 * 512
BINS = 128
NW = 32
SHARD = NPIX // NW
RW = 16
SROW = 2048 // NW
NSUB = 4
KEYS2 = 2 * BINS
LHIST = 16 * KEYS2


def _keys_body(x_ref, t_ref, out_ref):
    x = x_ref[0]
    m = jnp.max(x, axis=0, keepdims=True)
    e = jnp.exp(x - m)
    p = e / jnp.sum(e, axis=0, keepdims=True)
    t = t_ref[...]
    cls = lax.broadcasted_iota(jnp.int32, (NCLASS, 1, 1), 0)
    gt = t == cls
    a = jnp.where(gt, 1.0 - p, p)
    b = jnp.minimum((a * BINS).astype(jnp.int32), BINS - 1)
    out_ref[...] = jnp.where(gt, b + BINS, b)


def _keys(x, t):
    return pl.pallas_call(
        _keys_body,
        grid=(4, 512 // RW),
        in_specs=[
            pl.BlockSpec((1, NCLASS, RW, 512), lambda b, j: (b, 0, j, 0)),
            pl.BlockSpec((1, RW, 512), lambda b, j: (b, j, 0)),
        ],
        out_specs=pl.BlockSpec(
            (NCLASS, RW, 512), lambda b, j: (0, b * (512 // RW) + j, 0)),
        out_shape=jax.ShapeDtypeStruct((NCLASS, 2048, 512), jnp.int32),
        compiler_params=pltpu.CompilerParams(
            dimension_semantics=("parallel", "parallel")),
    )(x, t)


def _sc_hist_body(keys_hbm, hist_hbm, kb0, kb1, hist16, red, sem0, sem1):
    wid = lax.axis_index("s") * 2 + lax.axis_index("c")
    base = wid * SROW
    lane_off = lax.broadcasted_iota(jnp.int32, (16,), 0) * KEYS2
    ones = jnp.ones((16,), jnp.int32)
    zeros = jnp.zeros((16,), jnp.int32)

    def start(c, kb, sem):
        pltpu.async_copy(keys_hbm.at[c, pl.ds(base, SROW)], kb, sem)

    def wait(c, kb, sem):
        pltpu.make_async_copy(keys_hbm.at[c, pl.ds(base, SROW)], kb, sem).wait()

    def process(c, kb):
        def zero_step(i, _):
            for u in range(16):
                hist16[pl.ds((i * 16 + u) * 16, 16)] = zeros
            return ()
        lax.fori_loop(0, NSUB * LHIST // (16 * 16), zero_step, ())

        def scat_row(r, _):
            def scat_step(i, _):
                ks = [kb[r, pl.ds((i * 8 + u) * 16, 16)] for u in range(8)]
                idxs = [ks[u] + lane_off + (u % NSUB) * LHIST
                        for u in range(8)]
                for u in range(8):
                    plsc.addupdate_scatter(hist16, [idxs[u]], ones)
                return ()
            lax.fori_loop(0, 512 // (16 * 8), scat_step, ())
            return ()
        lax.fori_loop(0, SROW, scat_row, ())

        def red_step(j, _):
            vals = [hist16[pl.ds(t * KEYS2 + j * 16, 16)]
                    for t in range(NSUB * 16)]
            while len(vals) > 1:
                vals = [vals[t] + vals[t + 1] for t in range(0, len(vals), 2)]
            red[pl.ds(j * 16, 16)] = vals[0]
            return ()
        lax.fori_loop(0, KEYS2 // 16, red_step, ())

        pltpu.sync_copy(red, hist_hbm.at[c, wid])

    start(0, kb0, sem0)

    def per_class(c, _):
        @pl.when(c % 2 == 0)
        def _():
            wait(c, kb0, sem0)
            @pl.when(c + 1 < NCLASS)
            def _():
                start(c + 1, kb1, sem1)
            process(c, kb0)

        @pl.when(c % 2 == 1)
        def _():
            wait(c, kb1, sem1)
            @pl.when(c + 1 < NCLASS)
            def _():
                start(c + 1, kb0, sem0)
            process(c, kb1)

        return ()

    lax.fori_loop(0, NCLASS, per_class, ())


@functools.lru_cache(maxsize=1)
def _sc_hist_kernel():
    return pl.kernel(
        _sc_hist_body,
        mesh=plsc.VectorSubcoreMesh(core_axis_name="c", subcore_axis_name="s"),
        out_type=jax.ShapeDtypeStruct((NCLASS, NW, KEYS2), jnp.int32),
        scratch_types=[
            pltpu.VMEM((SROW, 512), jnp.int32),
            pltpu.VMEM((SROW, 512), jnp.int32),
            pltpu.VMEM((NSUB * LHIST,), jnp.int32),
            pltpu.VMEM((KEYS2,), jnp.int32),
            pltpu.SemaphoreType.DMA,
            pltpu.SemaphoreType.DMA,
        ],
        compiler_params=pltpu.CompilerParams(needs_layout_passes=False),
    )


def _sc_hist(keys):
    return _sc_hist_kernel()(keys)


def _strict_tri(n, lower=False):
    r = lax.broadcasted_iota(jnp.int32, (n, n), 0)
    col = lax.broadcasted_iota(jnp.int32, (n, n), 1)
    return ((r > col) if lower else (r < col)).astype(jnp.float32)


def _finish_body(hist_ref, out_ref):
    h = hist_ref[...].astype(jnp.float32)
    n2 = jnp.sum(h, axis=1)
    nn = n2[:, :BINS] + n2[:, BINS:]
    pp = n2[:, BINS:]
    tri = _strict_tri(BINS)
    aex = jnp.dot(nn, tri, preferred_element_type=jnp.float32)
    pex = jnp.dot(pp, tri, preferred_element_type=jnp.float32)
    nc = jnp.sum(nn, axis=1, keepdims=True)
    g = jnp.sum(pp, axis=1, keepdims=True)
    k = nc - aex
    s = g - pex
    u = g + k - s
    j = jnp.where(k > 0.5, 1.0 - (g - s) / jnp.maximum(u, 1.0), 0.0)
    val = (jnp.sum(j) - 0.5 * NCLASS) / (BINS * NCLASS)
    out_ref[...] = val.reshape(1, 1)


def _finish(hist):
    return pl.pallas_call(
        _finish_body,
        out_shape=jax.ShapeDtypeStruct((1, 1), jnp.float32),
    )(hist)


def kernel(input, target):
    keys = _keys(input, target)
    hist = _sc_hist(keys)
    return _finish(hist)[0, 0]


# speedup vs baseline: 1.1573x; 1.1089x over previous
import functools

import jax
import jax.numpy as jnp
from jax import lax
from jax.experimental import pallas as pl
from jax.experimental.pallas import tpu as pltpu
from jax.experimental.pallas import tpu_sc as plsc

NCLASS = 19
NPIX = 4 * 512 * 512
BINS = 128
NW = 32
SHARD = NPIX // NW
RW = 16
HROWS = 1024
SROW = HROWS // NW
NSUB = 4
KEYS2 = 2 * BINS
LHIST = 16 * KEYS2


def _keys_body(x_ref, t_ref, out_ref):
    x = x_ref[0]
    m = jnp.max(x, axis=0, keepdims=True)
    e = jnp.exp(x - m)
    p = e / jnp.sum(e, axis=0, keepdims=True)
    t = t_ref[...]
    cls = lax.broadcasted_iota(jnp.int32, (NCLASS, 1, 1), 0)
    gt = t == cls
    a = jnp.where(gt, 1.0 - p, p)
    b = jnp.minimum((a * BINS).astype(jnp.int32), BINS - 1)
    out_ref[...] = jnp.where(gt, b + BINS, b)


@functools.lru_cache(maxsize=2)
def _keys_kernel(b0):
    return pl.pallas_call(
        _keys_body,
        grid=(2, 512 // RW),
        in_specs=[
            pl.BlockSpec((1, NCLASS, RW, 512), lambda b, j: (b0 + b, 0, j, 0)),
            pl.BlockSpec((1, RW, 512), lambda b, j: (b0 + b, j, 0)),
        ],
        out_specs=pl.BlockSpec(
            (NCLASS, RW, 512), lambda b, j: (0, b * (512 // RW) + j, 0)),
        out_shape=jax.ShapeDtypeStruct((NCLASS, HROWS, 512), jnp.int32),
        compiler_params=pltpu.CompilerParams(
            dimension_semantics=("parallel", "parallel")),
    )


def _sc_hist_body(keys_hbm, hist_hbm, kb0, kb1, hist16, red, sem0, sem1):
    wid = lax.axis_index("s") * 2 + lax.axis_index("c")
    base = wid * SROW
    lane_off = lax.broadcasted_iota(jnp.int32, (16,), 0) * KEYS2
    ones = jnp.ones((16,), jnp.int32)
    zeros = jnp.zeros((16,), jnp.int32)

    def start(c, kb, sem):
        pltpu.async_copy(keys_hbm.at[c, pl.ds(base, SROW)], kb, sem)

    def wait(c, kb, sem):
        pltpu.make_async_copy(keys_hbm.at[c, pl.ds(base, SROW)], kb, sem).wait()

    def process(c, kb):
        def zero_step(i, _):
            for u in range(16):
                hist16[pl.ds((i * 16 + u) * 16, 16)] = zeros
            return ()
        lax.fori_loop(0, NSUB * LHIST // (16 * 16), zero_step, ())

        def scat_row(r, _):
            def scat_step(i, _):
                ks = [kb[r, pl.ds((i * 8 + u) * 16, 16)] for u in range(8)]
                idxs = [ks[u] + lane_off + (u % NSUB) * LHIST
                        for u in range(8)]
                for u in range(8):
                    plsc.addupdate_scatter(hist16, [idxs[u]], ones)
                return ()
            lax.fori_loop(0, 512 // (16 * 8), scat_step, ())
            return ()
        lax.fori_loop(0, SROW, scat_row, ())

        def red_step(j, _):
            vals = [hist16[pl.ds(t * KEYS2 + j * 16, 16)]
                    for t in range(NSUB * 16)]
            while len(vals) > 1:
                vals = [vals[t] + vals[t + 1] for t in range(0, len(vals), 2)]
            red[pl.ds(j * 16, 16)] = vals[0]
            return ()
        lax.fori_loop(0, KEYS2 // 16, red_step, ())

        pltpu.sync_copy(red, hist_hbm.at[c, wid])

    start(0, kb0, sem0)

    def per_class(c, _):
        @pl.when(c % 2 == 0)
        def _():
            wait(c, kb0, sem0)
            @pl.when(c + 1 < NCLASS)
            def _():
                start(c + 1, kb1, sem1)
            process(c, kb0)

        @pl.when(c % 2 == 1)
        def _():
            wait(c, kb1, sem1)
            @pl.when(c + 1 < NCLASS)
            def _():
                start(c + 1, kb0, sem0)
            process(c, kb1)

        return ()

    lax.fori_loop(0, NCLASS, per_class, ())


@functools.lru_cache(maxsize=1)
def _sc_hist_kernel():
    return pl.kernel(
        _sc_hist_body,
        mesh=plsc.VectorSubcoreMesh(core_axis_name="c", subcore_axis_name="s"),
        out_type=jax.ShapeDtypeStruct((NCLASS, NW, KEYS2), jnp.int32),
        scratch_types=[
            pltpu.VMEM((SROW, 512), jnp.int32),
            pltpu.VMEM((SROW, 512), jnp.int32),
            pltpu.VMEM((NSUB * LHIST,), jnp.int32),
            pltpu.VMEM((KEYS2,), jnp.int32),
            pltpu.SemaphoreType.DMA,
            pltpu.SemaphoreType.DMA,
        ],
        compiler_params=pltpu.CompilerParams(needs_layout_passes=False),
    )


def _sc_hist(keys):
    return _sc_hist_kernel()(keys)


def _strict_tri(n, lower=False):
    r = lax.broadcasted_iota(jnp.int32, (n, n), 0)
    col = lax.broadcasted_iota(jnp.int32, (n, n), 1)
    return ((r > col) if lower else (r < col)).astype(jnp.float32)


def _finish_body(hist_ref, out_ref):
    h = hist_ref[...].astype(jnp.float32)
    n2 = jnp.sum(h, axis=1)
    nn = n2[:, :BINS] + n2[:, BINS:]
    pp = n2[:, BINS:]
    tri = _strict_tri(BINS)
    aex = jnp.dot(nn, tri, preferred_element_type=jnp.float32)
    pex = jnp.dot(pp, tri, preferred_element_type=jnp.float32)
    nc = jnp.sum(nn, axis=1, keepdims=True)
    g = jnp.sum(pp, axis=1, keepdims=True)
    k = nc - aex
    s = g - pex
    u = g + k - s
    j = jnp.where(k > 0.5, 1.0 - (g - s) / jnp.maximum(u, 1.0), 0.0)
    val = (jnp.sum(j) - 0.5 * NCLASS) / (BINS * NCLASS)
    out_ref[...] = val.reshape(1, 1)


def _finish(hist):
    return pl.pallas_call(
        _finish_body,
        out_shape=jax.ShapeDtypeStruct((1, 1), jnp.float32),
    )(hist)


def kernel(input, target):
    k0 = _keys_kernel(0)(input, target)
    h0 = _sc_hist(k0)
    k1 = _keys_kernel(2)(input, target)
    h1 = _sc_hist(k1)
    return _finish(h0 + h1)[0, 0]


# speedup vs baseline: 1.1578x; 1.0004x over previous
import functools

import jax
import jax.numpy as jnp
from jax import lax
from jax.experimental import pallas as pl
from jax.experimental.pallas import tpu as pltpu
from jax.experimental.pallas import tpu_sc as plsc

NCLASS = 19
BINS = 128
NW = 32
RW = 16
HROWS = 1024
SROW = HROWS // NW
NSUB = 4
KEYS2 = 2 * BINS
LHIST = 16 * KEYS2


def _keys_body(x_ref, t_ref, out_ref):
    x = x_ref[0]
    m = jnp.max(x, axis=0, keepdims=True)
    e = jnp.exp(x - m)
    p = e / jnp.sum(e, axis=0, keepdims=True)
    t = t_ref[...]
    cls = lax.broadcasted_iota(jnp.int32, (NCLASS, 1, 1), 0)
    gt = t == cls
    a = jnp.where(gt, 1.0 - p, p)
    b = jnp.minimum((a * BINS).astype(jnp.int32), BINS - 1)
    out_ref[...] = jnp.where(gt, b + BINS, b)


@functools.lru_cache(maxsize=2)
def _keys_kernel(b0):
    return pl.pallas_call(
        _keys_body,
        grid=(2, 512 // RW),
        in_specs=[
            pl.BlockSpec((1, NCLASS, RW, 512), lambda b, j: (b0 + b, 0, j, 0)),
            pl.BlockSpec((1, RW, 512), lambda b, j: (b0 + b, j, 0)),
        ],
        out_specs=pl.BlockSpec(
            (NCLASS, RW, 512), lambda b, j: (0, b * (512 // RW) + j, 0)),
        out_shape=jax.ShapeDtypeStruct((NCLASS, HROWS, 512), jnp.int32),
        compiler_params=pltpu.CompilerParams(
            dimension_semantics=("parallel", "parallel")),
    )


def _sc_hist_body(keys_hbm, hist_hbm, kb0, kb1, hist16, red, sem0, sem1):
    wid = lax.axis_index("s") * 2 + lax.axis_index("c")
    base = wid * SROW
    lane_off = lax.broadcasted_iota(jnp.int32, (16,), 0) * KEYS2
    ones = jnp.ones((16,), jnp.int32)
    zeros = jnp.zeros((16,), jnp.int32)

    def start(c, kb, sem):
        pltpu.async_copy(keys_hbm.at[c, pl.ds(base, SROW)], kb, sem)

    def wait(c, kb, sem):
        pltpu.make_async_copy(keys_hbm.at[c, pl.ds(base, SROW)], kb, sem).wait()

    def process(c, kb):
        def zero_step(i, _):
            for u in range(16):
                hist16[pl.ds((i * 16 + u) * 16, 16)] = zeros
            return ()
        lax.fori_loop(0, NSUB * LHIST // (16 * 16), zero_step, ())

        def scat_row(r, _):
            def scat_step(i, _):
                ks = [kb[r, pl.ds((i * 8 + u) * 16, 16)] for u in range(8)]
                idxs = [ks[u] + lane_off + (u % NSUB) * LHIST
                        for u in range(8)]
                for u in range(8):
                    plsc.addupdate_scatter(hist16, [idxs[u]], ones)
                return ()
            lax.fori_loop(0, 512 // (16 * 8), scat_step, ())
            return ()
        lax.fori_loop(0, SROW, scat_row, ())

        def red_step(j, _):
            vals = [hist16[pl.ds(t * KEYS2 + j * 16, 16)]
                    for t in range(NSUB * 16)]
            while len(vals) > 1:
                vals = [vals[t] + vals[t + 1] for t in range(0, len(vals), 2)]
            red[pl.ds(j * 16, 16)] = vals[0]
            return ()
        lax.fori_loop(0, KEYS2 // 16, red_step, ())

        pltpu.sync_copy(red, hist_hbm.at[c, wid])

    start(0, kb0, sem0)

    def per_class(c, _):
        @pl.when(c % 2 == 0)
        def _():
            wait(c, kb0, sem0)
            @pl.when(c + 1 < NCLASS)
            def _():
                start(c + 1, kb1, sem1)
            process(c, kb0)

        @pl.when(c % 2 == 1)
        def _():
            wait(c, kb1, sem1)
            @pl.when(c + 1 < NCLASS)
            def _():
                start(c + 1, kb0, sem0)
            process(c, kb1)

        return ()

    lax.fori_loop(0, NCLASS, per_class, ())


@functools.lru_cache(maxsize=1)
def _sc_hist_kernel():
    return pl.kernel(
        _sc_hist_body,
        mesh=plsc.VectorSubcoreMesh(core_axis_name="c", subcore_axis_name="s"),
        out_type=jax.ShapeDtypeStruct((NCLASS, NW, KEYS2), jnp.int32),
        scratch_types=[
            pltpu.VMEM((SROW, 512), jnp.int32),
            pltpu.VMEM((SROW, 512), jnp.int32),
            pltpu.VMEM((NSUB * LHIST,), jnp.int32),
            pltpu.VMEM((KEYS2,), jnp.int32),
            pltpu.SemaphoreType.DMA,
            pltpu.SemaphoreType.DMA,
        ],
        compiler_params=pltpu.CompilerParams(needs_layout_passes=False),
    )


def _sc_hist(keys):
    return _sc_hist_kernel()(keys)


def _strict_tri(n):
    r = lax.broadcasted_iota(jnp.int32, (n, n), 0)
    col = lax.broadcasted_iota(jnp.int32, (n, n), 1)
    return (r < col).astype(jnp.float32)


def _finish_body(hist_ref, out_ref):
    h = hist_ref[...].astype(jnp.float32)
    n2 = jnp.sum(h, axis=1)
    nn = n2[:, :BINS] + n2[:, BINS:]
    pp = n2[:, BINS:]
    tri = _strict_tri(BINS)
    aex = jnp.dot(nn, tri, preferred_element_type=jnp.float32)
    pex = jnp.dot(pp, tri, preferred_element_type=jnp.float32)
    nc = jnp.sum(nn, axis=1, keepdims=True)
    g = jnp.sum(pp, axis=1, keepdims=True)
    k = nc - aex
    s = g - pex
    u = g + k - s
    j = jnp.where(k > 0.5, 1.0 - (g - s) / jnp.maximum(u, 1.0), 0.0)
    val = (jnp.sum(j) - 0.5 * NCLASS) / (BINS * NCLASS)
    out_ref[...] = val.reshape(1, 1)


def _finish(hist):
    return pl.pallas_call(
        _finish_body,
        out_shape=jax.ShapeDtypeStruct((1, 1), jnp.float32),
    )(hist)


def kernel(input, target):
    k0 = _keys_kernel(0)(input, target)
    h0 = _sc_hist(k0)
    k1 = _keys_kernel(2)(input, target)
    h1 = _sc_hist(k1)
    return _finish(h0 + h1)[0, 0]
